# baseline (speedup 1.0000x reference)
import functools

import jax
import jax.numpy as jnp
from jax import lax
from jax.experimental import pallas as pl
from jax.experimental.pallas import tpu as pltpu
from jax.experimental.pallas import tpu_sc as plsc

B = 16384
NI = 13
NC = 26
V = 10000
D = 16
R = B * NC

NW = 32
ROWS_PW = R // NW
GSZ = 128
CH = 1664
GP_CH = CH // GSZ
NCHUNK = ROWS_PW // CH
EPS = 1e-5


SMP_CH = CH // NC
EW = NC * D
EWP = 512


def _sc_gather(tab_flat, idx2d):
    mesh = plsc.VectorSubcoreMesh(core_axis_name="c", subcore_axis_name="s")

    @functools.partial(
        pl.kernel,
        out_type=jax.ShapeDtypeStruct((R, D), jnp.float32),
        mesh=mesh,
        scratch_types=[
            pltpu.VMEM((2, GP_CH, GSZ), jnp.int32),
            pltpu.VMEM((2, CH, D), jnp.float32),
            pltpu.SemaphoreType.DMA((2,)),
            pltpu.SemaphoreType.DMA((2,)),
            pltpu.SemaphoreType.DMA((2,)),
        ],
        compiler_params=pltpu.CompilerParams(use_tc_tiling_on_sc=False),
    )
    def k(tab_hbm, idx_hbm, out_hbm, idx_v, rows_v, sem_i, sem_g, sem_o):
        wid = lax.axis_index("s") * 2 + lax.axis_index("c")
        g0 = wid * (ROWS_PW // GSZ)

        def start_idx(ci, b):
            return pltpu.async_copy(
                idx_hbm.at[pl.ds(g0 + ci * GP_CH, GP_CH)],
                idx_v.at[b], sem_i.at[b])

        def start_gathers(b):
            return [
                pltpu.async_copy(
                    tab_hbm.at[idx_v.at[b, j]],
                    rows_v.at[b, pl.ds(j * GSZ, GSZ)],
                    sem_g.at[b])
                for j in range(GP_CH)
            ]

        out_cp = [None, None]
        idx_cp = start_idx(0, 0)
        for ci in range(NCHUNK):
            b = ci % 2
            idx_cp.wait()
            if out_cp[b] is not None:
                out_cp[b].wait()
            gcs = start_gathers(b)
            if ci + 1 < NCHUNK:
                idx_cp = start_idx(ci + 1, 1 - b)
            for c in gcs:
                c.wait()
            out_cp[b] = pltpu.async_copy(
                rows_v.at[b],
                out_hbm.at[pl.ds((g0 + ci * GP_CH) * GSZ, CH)],
                sem_o.at[b])
        for cp in out_cp:
            if cp is not None:
                cp.wait()

    return k(tab_flat, idx2d)


def _dot(a, b, prefer=jnp.float32):
    return jnp.dot(a, b, preferred_element_type=prefer)


def _bn_fold(stat_ref, g_ref, be_ref, n):
    m = stat_ref[0:1, :] * (1.0 / B)
    v = stat_ref[1:2, :] * (1.0 / B) - m * m
    a = g_ref[...] * lax.rsqrt(v + EPS)
    c = be_ref[...] - a * m
    return a, c


def _abc_pass(cont, emb, cat, w1c, w1e, b1, kmat, emat,
              w2, g1, be1, b2, w3, g2, be2, b3, tb=512):
    nb = B // tb
    dpd = EW
    n1, n2, n3 = w2.shape[0], w3.shape[0], w3.shape[1]

    def body(cont_ref, emb_ref, cat_ref, w1c_ref, w1e_ref, b1_ref, km_ref,
             em_ref, w2_ref, g1_ref, be1_ref, b2_ref, w3_ref, g2_ref,
             be2_ref, b3_ref,
             y3_ref, cross_ref, st3_ref,
             y1s, y2s, crs, st1, st2, w2f, w3f, b2f, b3f):
        p = pl.program_id(0)
        j = pl.program_id(1)
        r = pl.ds(j * tb, tb)

        @pl.when(p == 0)
        def _phase0():
            @pl.when(j == 0)
            def _():
                st1[...] = jnp.zeros_like(st1)

            mask = (cat_ref[...] != 0).astype(jnp.bfloat16)
            emb_t = emb_ref[...] * _dot(mask, em_ref[...]).astype(
                jnp.bfloat16)
            y = (_dot(cont_ref[...].astype(jnp.bfloat16), w1c_ref[...])
                 + _dot(emb_t, w1e_ref[...]) + b1_ref[...])
            y = jnp.maximum(y, 0.0)
            y1s[r, :] = y.astype(jnp.bfloat16)
            st1[0:1, :] += jnp.sum(y, axis=0, keepdims=True)
            st1[1:2, :] += jnp.sum(y * y, axis=0, keepdims=True)
            es = _dot(emb_t, km_ref[...])
            ess = _dot(emb_t * emb_t, km_ref[...])
            crs[r, :] = (0.5 * (es * es - ess)).astype(jnp.bfloat16)

        @pl.when(p == 1)
        def _phase1():
            @pl.when(j == 0)
            def _():
                a, c = _bn_fold(st1, g1_ref, be1_ref, n1)
                w2f[...] = (jnp.broadcast_to(a.reshape(n1, 1), w2_ref.shape)
                            * w2_ref[...]).astype(jnp.bfloat16)
                b2f[0:1, :] = _dot(c, w2_ref[...]) + b2_ref[...]
                st2[...] = jnp.zeros_like(st2)

            y = _dot(y1s[r, :], w2f[...]) + b2f[0:1, :]
            y = jnp.maximum(y, 0.0)
            y2s[r, :] = y.astype(jnp.bfloat16)
            st2[0:1, :] += jnp.sum(y, axis=0, keepdims=True)
            st2[1:2, :] += jnp.sum(y * y, axis=0, keepdims=True)

        @pl.when(p == 2)
        def _phase2():
            @pl.when(j == 0)
            def _():
                a, c = _bn_fold(st2, g2_ref, be2_ref, n2)
                w3f[...] = (jnp.broadcast_to(a.reshape(n2, 1), w3_ref.shape)
                            * w3_ref[...]).astype(jnp.bfloat16)
                b3f[0:1, :] = _dot(c, w3_ref[...]) + b3_ref[...]
                st3_ref[...] = jnp.zeros_like(st3_ref)

            y = _dot(y2s[r, :], w3f[...]) + b3f[0:1, :]
            y = jnp.maximum(y, 0.0)
            y3_ref[...] = y
            st3_ref[0:1, :] += jnp.sum(y, axis=0, keepdims=True)
            st3_ref[1:2, :] += jnp.sum(y * y, axis=0, keepdims=True)
            cross_ref[...] = crs[r, :].astype(jnp.float32)

    first = lambda p, j: (jnp.where(p == 0, j, 0), 0)
    const = lambda p, j: (0, 0)
    return pl.pallas_call(
        body,
        grid=(3, nb),
        in_specs=[
            pl.BlockSpec((tb, NI), first),
            pl.BlockSpec((tb, dpd), first),
            pl.BlockSpec((tb, NC), first),
            pl.BlockSpec((NI, n1), const),
            pl.BlockSpec((dpd, n1), const),
            pl.BlockSpec((1, n1), const),
            pl.BlockSpec((dpd, D), const),
            pl.BlockSpec((NC, dpd), const),
            pl.BlockSpec((n1, n2), const),
            pl.BlockSpec((1, n1), const),
            pl.BlockSpec((1, n1), const),
            pl.BlockSpec((1, n2), const),
            pl.BlockSpec((n2, n3), const),
            pl.BlockSpec((1, n2), const),
            pl.BlockSpec((1, n2), const),
            pl.BlockSpec((1, n3), const),
        ],
        out_specs=[
            pl.BlockSpec((tb, n3), lambda p, j: (j, 0)),
            pl.BlockSpec((tb, D), lambda p, j: (j, 0)),
            pl.BlockSpec((8, n3), const),
        ],
        out_shape=[
            jax.ShapeDtypeStruct((B, n3), jnp.float32),
            jax.ShapeDtypeStruct((B, D), jnp.float32),
            jax.ShapeDtypeStruct((8, n3), jnp.float32),
        ],
        scratch_shapes=[
            pltpu.VMEM((B, n1), jnp.bfloat16),
            pltpu.VMEM((B, n2), jnp.bfloat16),
            pltpu.VMEM((B, D), jnp.bfloat16),
            pltpu.VMEM((8, n1), jnp.float32),
            pltpu.VMEM((8, n2), jnp.float32),
            pltpu.VMEM((n1, n2), jnp.bfloat16),
            pltpu.VMEM((n2, n3), jnp.bfloat16),
            pltpu.VMEM((8, n2), jnp.float32),
            pltpu.VMEM((8, n3), jnp.float32),
        ],
    )(cont, emb, cat, w1c, w1e, b1, kmat, emat,
      w2, g1, be1, b2, w3, g2, be2, b3)


def _def_pass(y3, cross, cont, w4a, w4lin, w4c, b4f,
              w5, g4, be4, b5, w6, g5, be5, b6, tb=4096):
    nb = B // tb
    n4, n5 = w5.shape[0], w5.shape[1]

    def body(y3_ref, cr_ref, cont_ref, wa_ref, wl_ref, wc_ref, b4_ref,
             w5_ref, g4_ref, be4_ref, b5_ref, w6_ref, g5_ref, be5_ref,
             b6_ref,
             out_ref,
             f1s, f2s, st4, st5, w5f, w6f, b5f, b6f):
        p = pl.program_id(0)
        j = pl.program_id(1)
        r = pl.ds(j * tb, tb)

        @pl.when(p == 0)
        def _phase0():
            @pl.when(j == 0)
            def _():
                st4[...] = jnp.zeros_like(st4)

            linl = jnp.sum(cont_ref[...], axis=1, keepdims=True)
            y = (_dot(y3_ref[...], wa_ref[...])
                 + linl * wl_ref[...]
                 + _dot(cr_ref[...], wc_ref[...])
                 + b4_ref[...])
            y = jnp.maximum(y, 0.0)
            f1s[r, :] = y
            st4[0:1, :] += jnp.sum(y, axis=0, keepdims=True)
            st4[1:2, :] += jnp.sum(y * y, axis=0, keepdims=True)

        @pl.when(p == 1)
        def _phase1():
            @pl.when(j == 0)
            def _():
                a, c = _bn_fold(st4, g4_ref, be4_ref, n4)
                w5f[...] = jnp.broadcast_to(a.reshape(n4, 1), w5_ref.shape) \
                    * w5_ref[...]
                b5f[0:1, :] = _dot(c, w5_ref[...]) + b5_ref[...]
                st5[...] = jnp.zeros_like(st5)

            y = _dot(f1s[r, :], w5f[...]) + b5f[0:1, :]
            y = jnp.maximum(y, 0.0)
            f2s[r, :] = y
            st5[0:1, :] += jnp.sum(y, axis=0, keepdims=True)
            st5[1:2, :] += jnp.sum(y * y, axis=0, keepdims=True)

        @pl.when(p == 2)
        def _phase2():
            @pl.when(j == 0)
            def _():
                a, c = _bn_fold(st5, g5_ref, be5_ref, n5)
                w6f[...] = jnp.broadcast_to(a.reshape(n5, 1), w6_ref.shape) \
                    * w6_ref[...]
                b6f[0:1, :] = _dot(c, w6_ref[...]) + b6_ref[...]

            z = _dot(f2s[r, :], w6f[...]) + b6f[0:1, :]
            out_ref[...] = 1.0 / (1.0 + jnp.exp(-z))

    first = lambda p, j: (jnp.where(p == 0, j, 0), 0)
    const = lambda p, j: (0, 0)
    return pl.pallas_call(
        body,
        grid=(3, nb),
        in_specs=[
            pl.BlockSpec((tb, 32), first),
            pl.BlockSpec((tb, D), first),
            pl.BlockSpec((tb, NI), first),
            pl.BlockSpec((32, n4), const),
            pl.BlockSpec((1, n4), const),
            pl.BlockSpec((D, n4), const),
            pl.BlockSpec((1, n4), const),
            pl.BlockSpec((n4, n5), const),
            pl.BlockSpec((1, n4), const),
            pl.BlockSpec((1, n4), const),
            pl.BlockSpec((1, n5), const),
            pl.BlockSpec((n5, 1), const),
            pl.BlockSpec((1, n5), const),
            pl.BlockSpec((1, n5), const),
            pl.BlockSpec((1, 1), const),
        ],
        out_specs=[pl.BlockSpec((tb, 1), lambda p, j: (j, 0))],
        out_shape=[jax.ShapeDtypeStruct((B, 1), jnp.float32)],
        scratch_shapes=[
            pltpu.VMEM((B, n4), jnp.float32),
            pltpu.VMEM((B, n5), jnp.float32),
            pltpu.VMEM((8, n4), jnp.float32),
            pltpu.VMEM((8, n5), jnp.float32),
            pltpu.VMEM((n4, n5), jnp.float32),
            pltpu.VMEM((n5, 1), jnp.float32),
            pltpu.VMEM((8, n5), jnp.float32),
            pltpu.VMEM((8, 1), jnp.float32),
        ],
    )(y3, cross, cont, w4a, w4lin, w4c, b4f,
      w5, g4, be4, b5, w6, g5, be5, b6)[0]


def kernel(cont, cat, params):
    tables = params["tables"]
    dnn = params["dnn"]
    fin = params["final"]

    tab_flat = tables.reshape(NC * V, D)
    cat32 = cat.astype(jnp.int32)
    idx = cat32 + (jnp.arange(NC, dtype=jnp.int32) * V)[None, :]
    idx2d = idx.reshape(R // GSZ, GSZ)
    emb = _sc_gather(tab_flat, idx2d).reshape(B, EW).astype(jnp.bfloat16)

    w1, b1 = dnn[0]["W"], dnn[0]["b"][None, :]
    kmat = jnp.tile(jnp.eye(D, dtype=jnp.float32), (NC, 1))
    emat = jnp.kron(jnp.eye(NC, dtype=jnp.float32),
                    jnp.ones((1, D), jnp.float32))

    y3, cross, st3 = _abc_pass(
        cont, emb, cat32,
        w1[:NI].astype(jnp.bfloat16), w1[NI:].astype(jnp.bfloat16), b1,
        kmat.astype(jnp.bfloat16), emat.astype(jnp.bfloat16),
        dnn[1]["W"], dnn[0]["gamma"][None, :], dnn[0]["beta"][None, :],
        dnn[1]["b"][None, :],
        dnn[2]["W"], dnn[1]["gamma"][None, :], dnn[1]["beta"][None, :],
        dnn[2]["b"][None, :])

    m3 = st3[0] / B
    v3 = st3[1] / B - m3 * m3
    a3 = dnn[2]["gamma"] * lax.rsqrt(v3 + EPS)
    c3 = dnn[2]["beta"] - a3 * m3
    w4, b4 = fin[0]["W"], fin[0]["b"]
    w4a = a3[:, None] * w4[:32]
    b4f = (c3 @ w4[:32] + b4)[None, :]

    out = _def_pass(
        y3, cross, cont, w4a, w4[32:33], w4[33:49], b4f,
        fin[1]["W"], fin[0]["gamma"][None, :], fin[0]["beta"][None, :],
        fin[1]["b"][None, :],
        fin[2]["W"], fin[1]["gamma"][None, :], fin[1]["beta"][None, :],
        fin[2]["b"][None, :])
    return out.reshape(-1)


# speedup vs baseline: 1.5682x; 1.5682x over previous
import functools

import jax
import jax.numpy as jnp
from jax import lax
from jax.experimental import pallas as pl
from jax.experimental.pallas import tpu as pltpu
from jax.experimental.pallas import tpu_sc as plsc

B = 16384
NI = 13
NC = 26
V = 10000
D = 16
R = B * NC

NW = 32
ROWS_PW = R // NW
GSZ = 128
CH = 1664
GP_CH = CH // GSZ
NCHUNK = ROWS_PW // CH
EPS = 1e-5


SMP_CH = CH // NC
EW = NC * D
EWP = 512


def _sc_gather(tab_flat, idx2d):
    mesh = plsc.VectorSubcoreMesh(core_axis_name="c", subcore_axis_name="s")

    @functools.partial(
        pl.kernel,
        out_type=jax.ShapeDtypeStruct((R, D), jnp.float32),
        mesh=mesh,
        scratch_types=[
            pltpu.VMEM((2, GP_CH, GSZ), jnp.int32),
            pltpu.VMEM((2, CH, D), jnp.float32),
            pltpu.SemaphoreType.DMA((2,)),
            pltpu.SemaphoreType.DMA((2,)),
            pltpu.SemaphoreType.DMA((2,)),
        ],
        compiler_params=pltpu.CompilerParams(use_tc_tiling_on_sc=False),
    )
    def k(tab_hbm, idx_hbm, out_hbm, idx_v, rows_v, sem_i, sem_g, sem_o):
        wid = lax.axis_index("s") * 2 + lax.axis_index("c")
        g0 = wid * (ROWS_PW // GSZ)

        def start_idx(ci, b):
            return pltpu.async_copy(
                idx_hbm.at[pl.ds(g0 + ci * GP_CH, GP_CH)],
                idx_v.at[b], sem_i.at[b])

        def start_gathers(b):
            return [
                pltpu.async_copy(
                    tab_hbm.at[idx_v.at[b, j]],
                    rows_v.at[b, pl.ds(j * GSZ, GSZ)],
                    sem_g.at[b])
                for j in range(GP_CH)
            ]

        out_cp = [None, None]
        idx_cp = start_idx(0, 0)
        for ci in range(NCHUNK):
            b = ci % 2
            idx_cp.wait()
            if out_cp[b] is not None:
                out_cp[b].wait()
            gcs = start_gathers(b)
            if ci + 1 < NCHUNK:
                idx_cp = start_idx(ci + 1, 1 - b)
            for c in gcs:
                c.wait()
            out_cp[b] = pltpu.async_copy(
                rows_v.at[b],
                out_hbm.at[pl.ds((g0 + ci * GP_CH) * GSZ, CH)],
                sem_o.at[b])
        for cp in out_cp:
            if cp is not None:
                cp.wait()

    return k(tab_flat, idx2d)


def _dot(a, b, prefer=jnp.float32):
    return jnp.dot(a, b, preferred_element_type=prefer)


def _bn_fold(stat_ref, g_ref, be_ref, n):
    m = stat_ref[0:1, :] * (1.0 / B)
    v = stat_ref[1:2, :] * (1.0 / B) - m * m
    a = g_ref[...] * lax.rsqrt(v + EPS)
    c = be_ref[...] - a * m
    return a, c


def _abc_pass(cont, emb, cat, w1c, w1e, b1, kmat, emat,
              w2, g1, be1, b2, w3, g2, be2, b3, tb=512):
    nb = B // tb
    dpd = EW
    n1, n2, n3 = w2.shape[0], w3.shape[0], w3.shape[1]

    def body(cont_ref, emb_ref, cat_ref, w1c_ref, w1e_ref, b1_ref, km_ref,
             em_ref, w2_ref, g1_ref, be1_ref, b2_ref, w3_ref, g2_ref,
             be2_ref, b3_ref,
             y3_ref, cross_ref, st3_ref,
             y1s, y2s, crs, st1, st2, w2f, w3f, b2f, b3f):
        p = pl.program_id(0)
        j = pl.program_id(1)
        r = pl.ds(j * tb, tb)

        @pl.when(p == 0)
        def _phase0():
            @pl.when(j == 0)
            def _():
                st1[...] = jnp.zeros_like(st1)

            mask = (cat_ref[...] != 0).astype(jnp.bfloat16)
            emb_t = emb_ref[...].astype(jnp.bfloat16) \
                * _dot(mask, em_ref[...]).astype(jnp.bfloat16)
            y = (_dot(cont_ref[...].astype(jnp.bfloat16), w1c_ref[...])
                 + _dot(emb_t, w1e_ref[...]) + b1_ref[...])
            y = jnp.maximum(y, 0.0)
            y1s[r, :] = y.astype(jnp.bfloat16)
            st1[0:1, :] += jnp.sum(y, axis=0, keepdims=True)
            st1[1:2, :] += jnp.sum(y * y, axis=0, keepdims=True)
            es = _dot(emb_t, km_ref[...])
            ess = _dot(emb_t * emb_t, km_ref[...])
            crs[r, :] = (0.5 * (es * es - ess)).astype(jnp.bfloat16)

        @pl.when(p == 1)
        def _phase1():
            @pl.when(j == 0)
            def _():
                a, c = _bn_fold(st1, g1_ref, be1_ref, n1)
                w2f[...] = (jnp.broadcast_to(a.reshape(n1, 1), w2_ref.shape)
                            * w2_ref[...]).astype(jnp.bfloat16)
                b2f[0:1, :] = _dot(c, w2_ref[...]) + b2_ref[...]
                st2[...] = jnp.zeros_like(st2)

            y = _dot(y1s[r, :], w2f[...]) + b2f[0:1, :]
            y = jnp.maximum(y, 0.0)
            y2s[r, :] = y.astype(jnp.bfloat16)
            st2[0:1, :] += jnp.sum(y, axis=0, keepdims=True)
            st2[1:2, :] += jnp.sum(y * y, axis=0, keepdims=True)

        @pl.when(p == 2)
        def _phase2():
            @pl.when(j == 0)
            def _():
                a, c = _bn_fold(st2, g2_ref, be2_ref, n2)
                w3f[...] = (jnp.broadcast_to(a.reshape(n2, 1), w3_ref.shape)
                            * w3_ref[...]).astype(jnp.bfloat16)
                b3f[0:1, :] = _dot(c, w3_ref[...]) + b3_ref[...]
                st3_ref[...] = jnp.zeros_like(st3_ref)

            y = _dot(y2s[r, :], w3f[...]) + b3f[0:1, :]
            y = jnp.maximum(y, 0.0)
            y3_ref[...] = y
            st3_ref[0:1, :] += jnp.sum(y, axis=0, keepdims=True)
            st3_ref[1:2, :] += jnp.sum(y * y, axis=0, keepdims=True)
            cross_ref[...] = crs[r, :].astype(jnp.float32)

    first = lambda p, j: (jnp.where(p == 0, j, 0), 0)
    const = lambda p, j: (0, 0)
    return pl.pallas_call(
        body,
        grid=(3, nb),
        in_specs=[
            pl.BlockSpec((tb, NI), first),
            pl.BlockSpec((tb, dpd), first),
            pl.BlockSpec((tb, NC), first),
            pl.BlockSpec((NI, n1), const),
            pl.BlockSpec((dpd, n1), const),
            pl.BlockSpec((1, n1), const),
            pl.BlockSpec((dpd, D), const),
            pl.BlockSpec((NC, dpd), const),
            pl.BlockSpec((n1, n2), const),
            pl.BlockSpec((1, n1), const),
            pl.BlockSpec((1, n1), const),
            pl.BlockSpec((1, n2), const),
            pl.BlockSpec((n2, n3), const),
            pl.BlockSpec((1, n2), const),
            pl.BlockSpec((1, n2), const),
            pl.BlockSpec((1, n3), const),
        ],
        out_specs=[
            pl.BlockSpec((tb, n3), lambda p, j: (j, 0)),
            pl.BlockSpec((tb, D), lambda p, j: (j, 0)),
            pl.BlockSpec((8, n3), const),
        ],
        out_shape=[
            jax.ShapeDtypeStruct((B, n3), jnp.float32),
            jax.ShapeDtypeStruct((B, D), jnp.float32),
            jax.ShapeDtypeStruct((8, n3), jnp.float32),
        ],
        scratch_shapes=[
            pltpu.VMEM((B, n1), jnp.bfloat16),
            pltpu.VMEM((B, n2), jnp.bfloat16),
            pltpu.VMEM((B, D), jnp.bfloat16),
            pltpu.VMEM((8, n1), jnp.float32),
            pltpu.VMEM((8, n2), jnp.float32),
            pltpu.VMEM((n1, n2), jnp.bfloat16),
            pltpu.VMEM((n2, n3), jnp.bfloat16),
            pltpu.VMEM((8, n2), jnp.float32),
            pltpu.VMEM((8, n3), jnp.float32),
        ],
    )(cont, emb, cat, w1c, w1e, b1, kmat, emat,
      w2, g1, be1, b2, w3, g2, be2, b3)


def _def_pass(y3, cross, cont, w4a, w4lin, w4c, b4f,
              w5, g4, be4, b5, w6, g5, be5, b6, tb=4096):
    nb = B // tb
    n4, n5 = w5.shape[0], w5.shape[1]

    def body(y3_ref, cr_ref, cont_ref, wa_ref, wl_ref, wc_ref, b4_ref,
             w5_ref, g4_ref, be4_ref, b5_ref, w6_ref, g5_ref, be5_ref,
             b6_ref,
             out_ref,
             f1s, f2s, st4, st5, w5f, w6f, b5f, b6f):
        p = pl.program_id(0)
        j = pl.program_id(1)
        r = pl.ds(j * tb, tb)

        @pl.when(p == 0)
        def _phase0():
            @pl.when(j == 0)
            def _():
                st4[...] = jnp.zeros_like(st4)

            linl = jnp.sum(cont_ref[...], axis=1, keepdims=True)
            y = (_dot(y3_ref[...], wa_ref[...])
                 + linl * wl_ref[...]
                 + _dot(cr_ref[...], wc_ref[...])
                 + b4_ref[...])
            y = jnp.maximum(y, 0.0)
            f1s[r, :] = y
            st4[0:1, :] += jnp.sum(y, axis=0, keepdims=True)
            st4[1:2, :] += jnp.sum(y * y, axis=0, keepdims=True)

        @pl.when(p == 1)
        def _phase1():
            @pl.when(j == 0)
            def _():
                a, c = _bn_fold(st4, g4_ref, be4_ref, n4)
                w5f[...] = jnp.broadcast_to(a.reshape(n4, 1), w5_ref.shape) \
                    * w5_ref[...]
                b5f[0:1, :] = _dot(c, w5_ref[...]) + b5_ref[...]
                st5[...] = jnp.zeros_like(st5)

            y = _dot(f1s[r, :], w5f[...]) + b5f[0:1, :]
            y = jnp.maximum(y, 0.0)
            f2s[r, :] = y
            st5[0:1, :] += jnp.sum(y, axis=0, keepdims=True)
            st5[1:2, :] += jnp.sum(y * y, axis=0, keepdims=True)

        @pl.when(p == 2)
        def _phase2():
            @pl.when(j == 0)
            def _():
                a, c = _bn_fold(st5, g5_ref, be5_ref, n5)
                w6f[...] = jnp.broadcast_to(a.reshape(n5, 1), w6_ref.shape) \
                    * w6_ref[...]
                b6f[0:1, :] = _dot(c, w6_ref[...]) + b6_ref[...]

            z = _dot(f2s[r, :], w6f[...]) + b6f[0:1, :]
            out_ref[...] = 1.0 / (1.0 + jnp.exp(-z))

    first = lambda p, j: (jnp.where(p == 0, j, 0), 0)
    const = lambda p, j: (0, 0)
    return pl.pallas_call(
        body,
        grid=(3, nb),
        in_specs=[
            pl.BlockSpec((tb, 32), first),
            pl.BlockSpec((tb, D), first),
            pl.BlockSpec((tb, NI), first),
            pl.BlockSpec((32, n4), const),
            pl.BlockSpec((1, n4), const),
            pl.BlockSpec((D, n4), const),
            pl.BlockSpec((1, n4), const),
            pl.BlockSpec((n4, n5), const),
            pl.BlockSpec((1, n4), const),
            pl.BlockSpec((1, n4), const),
            pl.BlockSpec((1, n5), const),
            pl.BlockSpec((n5, 1), const),
            pl.BlockSpec((1, n5), const),
            pl.BlockSpec((1, n5), const),
            pl.BlockSpec((1, 1), const),
        ],
        out_specs=[pl.BlockSpec((tb, 1), lambda p, j: (j, 0))],
        out_shape=[jax.ShapeDtypeStruct((B, 1), jnp.float32)],
        scratch_shapes=[
            pltpu.VMEM((B, n4), jnp.float32),
            pltpu.VMEM((B, n5), jnp.float32),
            pltpu.VMEM((8, n4), jnp.float32),
            pltpu.VMEM((8, n5), jnp.float32),
            pltpu.VMEM((n4, n5), jnp.float32),
            pltpu.VMEM((n5, 1), jnp.float32),
            pltpu.VMEM((8, n5), jnp.float32),
            pltpu.VMEM((8, 1), jnp.float32),
        ],
    )(y3, cross, cont, w4a, w4lin, w4c, b4f,
      w5, g4, be4, b5, w6, g5, be5, b6)[0]


def kernel(cont, cat, params):
    tables = params["tables"]
    dnn = params["dnn"]
    fin = params["final"]

    tab_flat = tables.reshape(NC * V, D)
    cat32 = cat.astype(jnp.int32)
    idx = cat32 + (jnp.arange(NC, dtype=jnp.int32) * V)[None, :]
    idx2d = idx.reshape(R // GSZ, GSZ)
    emb = _sc_gather(tab_flat, idx2d).reshape(B, EW)

    w1, b1 = dnn[0]["W"], dnn[0]["b"][None, :]
    kmat = jnp.tile(jnp.eye(D, dtype=jnp.float32), (NC, 1))
    emat = jnp.kron(jnp.eye(NC, dtype=jnp.float32),
                    jnp.ones((1, D), jnp.float32))

    y3, cross, st3 = _abc_pass(
        cont, emb, cat32,
        w1[:NI].astype(jnp.bfloat16), w1[NI:].astype(jnp.bfloat16), b1,
        kmat.astype(jnp.bfloat16), emat.astype(jnp.bfloat16),
        dnn[1]["W"], dnn[0]["gamma"][None, :], dnn[0]["beta"][None, :],
        dnn[1]["b"][None, :],
        dnn[2]["W"], dnn[1]["gamma"][None, :], dnn[1]["beta"][None, :],
        dnn[2]["b"][None, :])

    m3 = st3[0] / B
    v3 = st3[1] / B - m3 * m3
    a3 = dnn[2]["gamma"] * lax.rsqrt(v3 + EPS)
    c3 = dnn[2]["beta"] - a3 * m3
    w4, b4 = fin[0]["W"], fin[0]["b"]
    w4a = a3[:, None] * w4[:32]
    b4f = (c3 @ w4[:32] + b4)[None, :]

    out = _def_pass(
        y3, cross, cont, w4a, w4[32:33], w4[33:49], b4f,
        fin[1]["W"], fin[0]["gamma"][None, :], fin[0]["beta"][None, :],
        fin[1]["b"][None, :],
        fin[2]["W"], fin[1]["gamma"][None, :], fin[1]["beta"][None, :],
        fin[2]["b"][None, :])
    return out.reshape(-1)


# speedup vs baseline: 1.6284x; 1.0384x over previous
import functools

import jax
import jax.numpy as jnp
from jax import lax
from jax.experimental import pallas as pl
from jax.experimental.pallas import tpu as pltpu
from jax.experimental.pallas import tpu_sc as plsc

B = 16384
NI = 13
NC = 26
V = 10000
D = 16
R = B * NC

NW = 32
ROWS_PW = R // NW
GSZ = 128
CH = 1664
GP_CH = CH // GSZ
NCHUNK = ROWS_PW // CH
EPS = 1e-5


SMP_CH = CH // NC
EW = NC * D
EWP = 512


def _sc_gather(tab_flat, idx2d):
    mesh = plsc.VectorSubcoreMesh(core_axis_name="c", subcore_axis_name="s")

    @functools.partial(
        pl.kernel,
        out_type=jax.ShapeDtypeStruct((R, D), jnp.float32),
        mesh=mesh,
        scratch_types=[
            pltpu.VMEM((2, GP_CH, GSZ), jnp.int32),
            pltpu.VMEM((2, CH, D), jnp.float32),
            pltpu.SemaphoreType.DMA((2,)),
            pltpu.SemaphoreType.DMA((2,)),
            pltpu.SemaphoreType.DMA((2,)),
        ],
        compiler_params=pltpu.CompilerParams(use_tc_tiling_on_sc=False),
    )
    def k(tab_hbm, idx_hbm, out_hbm, idx_v, rows_v, sem_i, sem_g, sem_o):
        wid = lax.axis_index("s") * 2 + lax.axis_index("c")
        g0 = wid * (ROWS_PW // GSZ)

        def start_idx(ci, b):
            return pltpu.async_copy(
                idx_hbm.at[pl.ds(g0 + ci * GP_CH, GP_CH)],
                idx_v.at[b], sem_i.at[b])

        def start_gathers(b):
            return [
                pltpu.async_copy(
                    tab_hbm.at[idx_v.at[b, j]],
                    rows_v.at[b, pl.ds(j * GSZ, GSZ)],
                    sem_g.at[b])
                for j in range(GP_CH)
            ]

        out_cp = [None, None]
        idx_cp = start_idx(0, 0)
        for ci in range(NCHUNK):
            b = ci % 2
            idx_cp.wait()
            if out_cp[b] is not None:
                out_cp[b].wait()
            gcs = start_gathers(b)
            if ci + 1 < NCHUNK:
                idx_cp = start_idx(ci + 1, 1 - b)
            for c in gcs:
                c.wait()
            out_cp[b] = pltpu.async_copy(
                rows_v.at[b],
                out_hbm.at[pl.ds((g0 + ci * GP_CH) * GSZ, CH)],
                sem_o.at[b])
        for cp in out_cp:
            if cp is not None:
                cp.wait()

    return k(tab_flat, idx2d)


def _dot(a, b, prefer=jnp.float32):
    return jnp.dot(a, b, preferred_element_type=prefer)


def _bn_fold(stat_ref, g_ref, be_ref, n):
    m = stat_ref[0:1, :] * (1.0 / B)
    v = stat_ref[1:2, :] * (1.0 / B) - m * m
    a = g_ref[...] * lax.rsqrt(v + EPS)
    c = be_ref[...] - a * m
    return a, c


def _abc_pass(cont, emb, cat, w1c, w1e, b1, kmat, emat,
              w2, g1, be1, b2, w3, g2, be2, b3, tb=1024):
    nb = B // tb
    dpd = EW
    n1, n2, n3 = w2.shape[0], w3.shape[0], w3.shape[1]

    def body(cont_ref, emb_ref, cat_ref, w1c_ref, w1e_ref, b1_ref, km_ref,
             em_ref, w2_ref, g1_ref, be1_ref, b2_ref, w3_ref, g2_ref,
             be2_ref, b3_ref,
             y3_ref, cross_ref, st3_ref,
             y1s, y2s, st1, st2, w2f, w3f, b2f, b3f):
        p = pl.program_id(0)
        j = pl.program_id(1)
        r = pl.ds(j * tb, tb)

        @pl.when(p == 0)
        def _phase0():
            @pl.when(j == 0)
            def _():
                st1[...] = jnp.zeros_like(st1)

            mask = (cat_ref[...] != 0).astype(jnp.bfloat16)
            emb_t = emb_ref[...].astype(jnp.bfloat16) \
                * _dot(mask, em_ref[...]).astype(jnp.bfloat16)
            y = (_dot(cont_ref[...].astype(jnp.bfloat16), w1c_ref[...])
                 + _dot(emb_t, w1e_ref[...]) + b1_ref[...])
            y = jnp.maximum(y, 0.0)
            y1s[r, :] = y.astype(jnp.bfloat16)
            st1[0:1, :] += jnp.sum(y, axis=0, keepdims=True)
            st1[1:2, :] += jnp.sum(y * y, axis=0, keepdims=True)
            es = _dot(emb_t, km_ref[...])
            ess = _dot(emb_t * emb_t, km_ref[...])
            cross_ref[...] = 0.5 * (es * es - ess)

        @pl.when(p == 1)
        def _phase1():
            @pl.when(j == 0)
            def _():
                a, c = _bn_fold(st1, g1_ref, be1_ref, n1)
                w2f[...] = (jnp.broadcast_to(a.reshape(n1, 1), w2_ref.shape)
                            * w2_ref[...]).astype(jnp.bfloat16)
                b2f[0:1, :] = _dot(c, w2_ref[...]) + b2_ref[...]
                st2[...] = jnp.zeros_like(st2)

            y = _dot(y1s[r, :], w2f[...]) + b2f[0:1, :]
            y = jnp.maximum(y, 0.0)
            y2s[r, :] = y.astype(jnp.bfloat16)
            st2[0:1, :] += jnp.sum(y, axis=0, keepdims=True)
            st2[1:2, :] += jnp.sum(y * y, axis=0, keepdims=True)

        @pl.when(p == 2)
        def _phase2():
            @pl.when(j == 0)
            def _():
                a, c = _bn_fold(st2, g2_ref, be2_ref, n2)
                w3f[...] = (jnp.broadcast_to(a.reshape(n2, 1), w3_ref.shape)
                            * w3_ref[...]).astype(jnp.bfloat16)
                b3f[0:1, :] = _dot(c, w3_ref[...]) + b3_ref[...]
                st3_ref[...] = jnp.zeros_like(st3_ref)

            y = _dot(y2s[r, :], w3f[...]) + b3f[0:1, :]
            y = jnp.maximum(y, 0.0)
            y3_ref[...] = y
            st3_ref[0:1, :] += jnp.sum(y, axis=0, keepdims=True)
            st3_ref[1:2, :] += jnp.sum(y * y, axis=0, keepdims=True)

    first = lambda p, j: (jnp.where(p == 0, j, 0), 0)
    const = lambda p, j: (0, 0)
    return pl.pallas_call(
        body,
        grid=(3, nb),
        in_specs=[
            pl.BlockSpec((tb, NI), first),
            pl.BlockSpec((tb, dpd), first),
            pl.BlockSpec((tb, NC), first),
            pl.BlockSpec((NI, n1), const),
            pl.BlockSpec((dpd, n1), const),
            pl.BlockSpec((1, n1), const),
            pl.BlockSpec((dpd, D), const),
            pl.BlockSpec((NC, dpd), const),
            pl.BlockSpec((n1, n2), const),
            pl.BlockSpec((1, n1), const),
            pl.BlockSpec((1, n1), const),
            pl.BlockSpec((1, n2), const),
            pl.BlockSpec((n2, n3), const),
            pl.BlockSpec((1, n2), const),
            pl.BlockSpec((1, n2), const),
            pl.BlockSpec((1, n3), const),
        ],
        out_specs=[
            pl.BlockSpec((tb, n3), lambda p, j: (j, 0)),
            pl.BlockSpec((tb, D), lambda p, j: (jnp.where(p == 0, j, nb), 0)),
            pl.BlockSpec((8, n3), const),
        ],
        out_shape=[
            jax.ShapeDtypeStruct((B, n3), jnp.float32),
            jax.ShapeDtypeStruct((B + tb, D), jnp.float32),
            jax.ShapeDtypeStruct((8, n3), jnp.float32),
        ],
        scratch_shapes=[
            pltpu.VMEM((B, n1), jnp.bfloat16),
            pltpu.VMEM((B, n2), jnp.bfloat16),
            pltpu.VMEM((8, n1), jnp.float32),
            pltpu.VMEM((8, n2), jnp.float32),
            pltpu.VMEM((n1, n2), jnp.bfloat16),
            pltpu.VMEM((n2, n3), jnp.bfloat16),
            pltpu.VMEM((8, n2), jnp.float32),
            pltpu.VMEM((8, n3), jnp.float32),
        ],
    )(cont, emb, cat, w1c, w1e, b1, kmat, emat,
      w2, g1, be1, b2, w3, g2, be2, b3)


def _def_pass(y3, cross, cont, w4a, w4lin, w4c, b4f,
              w5, g4, be4, b5, w6, g5, be5, b6, tb=4096):
    nb = B // tb
    n4, n5 = w5.shape[0], w5.shape[1]

    def body(y3_ref, cr_ref, cont_ref, wa_ref, wl_ref, wc_ref, b4_ref,
             w5_ref, g4_ref, be4_ref, b5_ref, w6_ref, g5_ref, be5_ref,
             b6_ref,
             out_ref,
             f1s, f2s, st4, st5, w5f, w6f, b5f, b6f):
        p = pl.program_id(0)
        j = pl.program_id(1)
        r = pl.ds(j * tb, tb)

        @pl.when(p == 0)
        def _phase0():
            @pl.when(j == 0)
            def _():
                st4[...] = jnp.zeros_like(st4)

            linl = jnp.sum(cont_ref[...], axis=1, keepdims=True)
            y = (_dot(y3_ref[...], wa_ref[...])
                 + linl * wl_ref[...]
                 + _dot(cr_ref[...], wc_ref[...])
                 + b4_ref[...])
            y = jnp.maximum(y, 0.0)
            f1s[r, :] = y
            st4[0:1, :] += jnp.sum(y, axis=0, keepdims=True)
            st4[1:2, :] += jnp.sum(y * y, axis=0, keepdims=True)

        @pl.when(p == 1)
        def _phase1():
            @pl.when(j == 0)
            def _():
                a, c = _bn_fold(st4, g4_ref, be4_ref, n4)
                w5f[...] = jnp.broadcast_to(a.reshape(n4, 1), w5_ref.shape) \
                    * w5_ref[...]
                b5f[0:1, :] = _dot(c, w5_ref[...]) + b5_ref[...]
                st5[...] = jnp.zeros_like(st5)

            y = _dot(f1s[r, :], w5f[...]) + b5f[0:1, :]
            y = jnp.maximum(y, 0.0)
            f2s[r, :] = y
            st5[0:1, :] += jnp.sum(y, axis=0, keepdims=True)
            st5[1:2, :] += jnp.sum(y * y, axis=0, keepdims=True)

        @pl.when(p == 2)
        def _phase2():
            @pl.when(j == 0)
            def _():
                a, c = _bn_fold(st5, g5_ref, be5_ref, n5)
                w6f[...] = jnp.broadcast_to(a.reshape(n5, 1), w6_ref.shape) \
                    * w6_ref[...]
                b6f[0:1, :] = _dot(c, w6_ref[...]) + b6_ref[...]

            z = _dot(f2s[r, :], w6f[...]) + b6f[0:1, :]
            out_ref[...] = 1.0 / (1.0 + jnp.exp(-z))

    first = lambda p, j: (jnp.where(p == 0, j, 0), 0)
    const = lambda p, j: (0, 0)
    return pl.pallas_call(
        body,
        grid=(3, nb),
        in_specs=[
            pl.BlockSpec((tb, 32), first),
            pl.BlockSpec((tb, D), first),
            pl.BlockSpec((tb, NI), first),
            pl.BlockSpec((32, n4), const),
            pl.BlockSpec((1, n4), const),
            pl.BlockSpec((D, n4), const),
            pl.BlockSpec((1, n4), const),
            pl.BlockSpec((n4, n5), const),
            pl.BlockSpec((1, n4), const),
            pl.BlockSpec((1, n4), const),
            pl.BlockSpec((1, n5), const),
            pl.BlockSpec((n5, 1), const),
            pl.BlockSpec((1, n5), const),
            pl.BlockSpec((1, n5), const),
            pl.BlockSpec((1, 1), const),
        ],
        out_specs=[pl.BlockSpec((tb, 1), lambda p, j: (j, 0))],
        out_shape=[jax.ShapeDtypeStruct((B, 1), jnp.float32)],
        scratch_shapes=[
            pltpu.VMEM((B, n4), jnp.float32),
            pltpu.VMEM((B, n5), jnp.float32),
            pltpu.VMEM((8, n4), jnp.float32),
            pltpu.VMEM((8, n5), jnp.float32),
            pltpu.VMEM((n4, n5), jnp.float32),
            pltpu.VMEM((n5, 1), jnp.float32),
            pltpu.VMEM((8, n5), jnp.float32),
            pltpu.VMEM((8, 1), jnp.float32),
        ],
    )(y3, cross, cont, w4a, w4lin, w4c, b4f,
      w5, g4, be4, b5, w6, g5, be5, b6)[0]


def kernel(cont, cat, params):
    tables = params["tables"]
    dnn = params["dnn"]
    fin = params["final"]

    tab_flat = tables.reshape(NC * V, D)
    cat32 = cat.astype(jnp.int32)
    idx = cat32 + (jnp.arange(NC, dtype=jnp.int32) * V)[None, :]
    idx2d = idx.reshape(R // GSZ, GSZ)
    emb = _sc_gather(tab_flat, idx2d).reshape(B, EW)

    w1, b1 = dnn[0]["W"], dnn[0]["b"][None, :]
    kmat = jnp.tile(jnp.eye(D, dtype=jnp.float32), (NC, 1))
    emat = jnp.kron(jnp.eye(NC, dtype=jnp.float32),
                    jnp.ones((1, D), jnp.float32))

    y3, cross_pad, st3 = _abc_pass(
        cont, emb, cat32,
        w1[:NI].astype(jnp.bfloat16), w1[NI:].astype(jnp.bfloat16), b1,
        kmat.astype(jnp.bfloat16), emat.astype(jnp.bfloat16),
        dnn[1]["W"], dnn[0]["gamma"][None, :], dnn[0]["beta"][None, :],
        dnn[1]["b"][None, :],
        dnn[2]["W"], dnn[1]["gamma"][None, :], dnn[1]["beta"][None, :],
        dnn[2]["b"][None, :])

    m3 = st3[0] / B
    v3 = st3[1] / B - m3 * m3
    a3 = dnn[2]["gamma"] * lax.rsqrt(v3 + EPS)
    c3 = dnn[2]["beta"] - a3 * m3
    w4, b4 = fin[0]["W"], fin[0]["b"]
    w4a = a3[:, None] * w4[:32]
    b4f = (c3 @ w4[:32] + b4)[None, :]

    out = _def_pass(
        y3, cross_pad[:B], cont, w4a, w4[32:33], w4[33:49], b4f,
        fin[1]["W"], fin[0]["gamma"][None, :], fin[0]["beta"][None, :],
        fin[1]["b"][None, :],
        fin[2]["W"], fin[1]["gamma"][None, :], fin[1]["beta"][None, :],
        fin[2]["b"][None, :])
    return out.reshape(-1)
